# Initial kernel scaffold; baseline (speedup 1.0000x reference)
import jax
import jax.numpy as jnp
from jax.experimental import pallas as pl


def kernel(h, edge_index_0, edge_index_1, edge_index_2, W_0, W_1, W_2, al_0, al_1, al_2, ar_0, ar_1, ar_2, b_0, b_1, b_2, P1_w, P1_b, P2_w, Wp, bp):
    raise NotImplementedError("write your pallas kernel here")


# speedup vs baseline: 1.0237x; 1.0237x over previous
import functools

import jax
import jax.numpy as jnp
from jax.experimental import pallas as pl
from jax.experimental.pallas import tpu as pltpu

N = 10000
E = 160000
M = 3
IN_DIM = 128
HID = 64
HEADS = 4
OUT_DIM = 64
SEM_HID = 128

NBLK = 400


def _z_proj_body(h_ref, w_ref, z_ref):
    z_ref[0] = jnp.dot(h_ref[...], w_ref[0], preferred_element_type=jnp.float32)


def _z_proj(h, W_all):
    return pl.pallas_call(
        _z_proj_body,
        grid=(M, N // NBLK),
        in_specs=[
            pl.BlockSpec((NBLK, IN_DIM), lambda i, j: (j, 0)),
            pl.BlockSpec((1, IN_DIM, HEADS * HID), lambda i, j: (i, 0, 0)),
        ],
        out_specs=pl.BlockSpec((1, NBLK, HEADS * HID), lambda i, j: (i, j, 0)),
        out_shape=jax.ShapeDtypeStruct((M, N, HEADS * HID), jnp.float32),
    )(h, W_all)


def kernel(h, edge_index_0, edge_index_1, edge_index_2, W_0, W_1, W_2, al_0, al_1, al_2, ar_0, ar_1, ar_2, b_0, b_1, b_2, P1_w, P1_b, P2_w, Wp, bp):
    edges = [edge_index_0, edge_index_1, edge_index_2]
    als = [al_0, al_1, al_2]
    ars = [ar_0, ar_1, ar_2]
    bs = [b_0, b_1, b_2]
    W_all = jnp.stack([W_0, W_1, W_2])
    z_all = _z_proj(h, W_all)

    zp_list = []
    alpha_list = []
    for i in range(M):
        src, dst = edges[i][0], edges[i][1]
        z = z_all[i].reshape(N, HEADS, HID)
        el = (z * als[i][None]).sum(-1)
        er = (z * ars[i][None]).sum(-1)
        e = jax.nn.leaky_relu(el[src] + er[dst], 0.2)
        ex = jnp.exp(e)
        s = jax.ops.segment_sum(ex, dst, num_segments=N)
        a = ex / jnp.maximum(s[dst], 1e-9)
        o = jax.ops.segment_sum(a[:, :, None] * z[src], dst, num_segments=N)
        o = jax.nn.elu(o + bs[i].reshape(1, HEADS, HID))
        zp_list.append(o.reshape(N, HEADS * HID))
        alpha_list.append(a.mean(1))
    zp = jnp.stack(zp_list, axis=1)
    w = jax.nn.leaky_relu((zp @ P1_w + P1_b) @ P2_w, 0.01).mean(axis=0)
    beta = jax.nn.softmax(w, axis=0)
    atten = jnp.zeros((N, N), dtype=jnp.float32)
    for i in range(M):
        atten = atten.at[edges[i][0], edges[i][1]].add(alpha_list[i] * beta[i, 0])
    z_sem = (beta[None, :, :] * zp).sum(axis=1)
    out = z_sem @ Wp + bp
    return out, atten


# speedup vs baseline: 7.7105x; 7.5322x over previous
import functools

import jax
import jax.numpy as jnp
from jax import lax
from jax.experimental import pallas as pl
from jax.experimental.pallas import tpu as pltpu
from jax.experimental.pallas import tpu_sc as plsc

N = 10000
E = 160000
M = 3
IN_DIM = 128
HID = 64
HEADS = 4
OUT_DIM = 64
SEM_HID = 128

NBLK = 400

NC = 2
NS = 16
HPC = HEADS // NC
DPC = HPC * HID
EPS = E // NS
CHUNK = 128
NFULL = EPS // CHUNK
REM = EPS - NFULL * CHUNK
ZROWS = 640
ZROWS_L = N - 15 * ZROWS


def _z_proj_body(h_ref, w_ref, z_ref):
    z_ref[0, 0] = jnp.dot(h_ref[...], w_ref[0, 0], preferred_element_type=jnp.float32)


def _z_proj(h, W_all):
    return pl.pallas_call(
        _z_proj_body,
        grid=(M, NC, N // NBLK),
        in_specs=[
            pl.BlockSpec((NBLK, IN_DIM), lambda i, cc, j: (j, 0)),
            pl.BlockSpec((1, 1, IN_DIM, DPC), lambda i, cc, j: (i, cc, 0, 0)),
        ],
        out_specs=pl.BlockSpec((1, 1, NBLK, DPC), lambda i, cc, j: (i, cc, j, 0)),
        out_shape=jax.ShapeDtypeStruct((M, NC, N, DPC), jnp.float32),
    )(h, W_all)


def _scale_rows(rowbuf, a0, a1, nrows):
    for g in range(nrows // 16):
        av0 = a0[pl.ds(g * 16, 16)]
        av1 = a1[pl.ds(g * 16, 16)]
        for j in range(16):
            r = g * 16 + j
            c0 = av0[j]
            c1 = av1[j]
            for c in range(HID // 16):
                sl = pl.ds(c * 16, 16)
                rowbuf[r, sl] = rowbuf[r, sl] * c0
                sl2 = pl.ds(HID + c * 16, 16)
                rowbuf[r, sl2] = rowbuf[r, sl2] * c1


def _sc_o_body(z_cm, src_f, dst_f, a_f, zrows, o_cm, acc, idxbuf, didxbuf,
               a0buf, a1buf, rowbuf, idxbuf_r, didxbuf_r, a0buf_r, a1buf_r,
               rowbuf_r, sem):
    c = lax.axis_index("c")
    s = lax.axis_index("s")
    e_base = s * EPS
    sync = pltpu.sync_copy
    r0 = s * ZROWS

    for i in range(M):
        @pl.when(s < NS - 1)
        def _():
            sync(zrows.at[pl.ds(0, ZROWS)], acc.at[pl.ds(r0, ZROWS)])

        @pl.when(s == NS - 1)
        def _():
            sync(zrows.at[pl.ds(0, ZROWS_L)], acc.at[pl.ds(r0, ZROWS_L)])

        plsc.subcore_barrier()

        def do_chunk(e0, nrows, ibuf, dibuf, a0b, a1b, rb):
            sync(src_f.at[pl.ds(i * E + e0, nrows)], ibuf)
            sync(dst_f.at[pl.ds(i * E + e0, nrows)], dibuf)
            sync(a_f.at[pl.ds(((i * NC + c) * HPC + 0) * E + e0, nrows)], a0b)
            sync(a_f.at[pl.ds(((i * NC + c) * HPC + 1) * E + e0, nrows)], a1b)
            pltpu.async_copy(z_cm.at[i, c].at[ibuf], rb, sem).wait()
            _scale_rows(rb, a0b, a1b, nrows)
            sync(rb, acc.at[dibuf], add=True)

        def chunk_body(k, carry):
            do_chunk(e_base + k * CHUNK, CHUNK, idxbuf, didxbuf, a0buf,
                     a1buf, rowbuf)
            return carry

        lax.fori_loop(0, NFULL, chunk_body, 0)
        if REM:
            do_chunk(e_base + NFULL * CHUNK, REM, idxbuf_r, didxbuf_r,
                     a0buf_r, a1buf_r, rowbuf_r)
        plsc.subcore_barrier()

        @pl.when(s < NS - 1)
        def _():
            sync(acc.at[pl.ds(r0, ZROWS)], o_cm.at[i, c, pl.ds(r0, ZROWS)])

        @pl.when(s == NS - 1)
        def _():
            sync(acc.at[pl.ds(r0, ZROWS_L)], o_cm.at[i, c, pl.ds(r0, ZROWS_L)])

        plsc.subcore_barrier()


def _sc_o(z_cm, src_f, dst_f, a_f):
    zrows = jnp.zeros((ZROWS, DPC), jnp.float32)
    mesh = plsc.VectorSubcoreMesh(core_axis_name="c", subcore_axis_name="s")
    f = pl.kernel(
        _sc_o_body,
        out_type=jax.ShapeDtypeStruct((M, NC, N, DPC), jnp.float32),
        mesh=mesh,
        scratch_types=[
            pltpu.VMEM_SHARED((N, DPC), jnp.float32),
            pltpu.VMEM((CHUNK,), jnp.int32),
            pltpu.VMEM((CHUNK,), jnp.int32),
            pltpu.VMEM((CHUNK,), jnp.float32),
            pltpu.VMEM((CHUNK,), jnp.float32),
            pltpu.VMEM((CHUNK, DPC), jnp.float32),
            pltpu.VMEM((REM,), jnp.int32),
            pltpu.VMEM((REM,), jnp.int32),
            pltpu.VMEM((REM,), jnp.float32),
            pltpu.VMEM((REM,), jnp.float32),
            pltpu.VMEM((REM, DPC), jnp.float32),
            pltpu.SemaphoreType.DMA,
        ],
    )
    return f(z_cm, src_f, dst_f, a_f, zrows)


def kernel(h, edge_index_0, edge_index_1, edge_index_2, W_0, W_1, W_2, al_0, al_1, al_2, ar_0, ar_1, ar_2, b_0, b_1, b_2, P1_w, P1_b, P2_w, Wp, bp):
    edges = [edge_index_0, edge_index_1, edge_index_2]
    als = [al_0, al_1, al_2]
    ars = [ar_0, ar_1, ar_2]
    bs = [b_0, b_1, b_2]
    W_all = jnp.stack([W_0, W_1, W_2]).reshape(M, IN_DIM, NC, DPC).transpose(0, 2, 1, 3)
    z_cm = _z_proj(h, W_all)
    src_f = jnp.stack([e[0] for e in edges]).reshape(M * E)
    dst_f = jnp.stack([e[1] for e in edges]).reshape(M * E)

    a_list = []
    for i in range(M):
        src, dst = edges[i][0], edges[i][1]
        z = z_cm[i].reshape(NC, N, HPC, HID).transpose(0, 2, 1, 3).reshape(HEADS, N, HID)
        el = (z * als[i][:, None, :]).sum(-1)
        er = (z * ars[i][:, None, :]).sum(-1)
        e = jax.nn.leaky_relu(el[:, src] + er[:, dst], 0.2)
        ex = jnp.exp(e)
        ssum = jax.vmap(lambda x: jax.ops.segment_sum(x, dst, num_segments=N))(ex)
        a = ex / jnp.maximum(ssum[:, dst], 1e-9)
        a_list.append(a)
    a_hm = jnp.stack(a_list)
    a_f = a_hm.reshape(M * HEADS * E)

    o_cm = _sc_o(z_cm, src_f, dst_f, a_f)

    zp_list = []
    alpha_list = []
    for i in range(M):
        o = o_cm[i].reshape(NC, N, HPC, HID).transpose(1, 0, 2, 3)
        o = o.reshape(N, HEADS, HID)
        o = jax.nn.elu(o + bs[i].reshape(1, HEADS, HID))
        zp_list.append(o.reshape(N, HEADS * HID))
        alpha_list.append(a_hm[i].mean(0))
    zp = jnp.stack(zp_list, axis=1)
    w = jax.nn.leaky_relu((zp @ P1_w + P1_b) @ P2_w, 0.01).mean(axis=0)
    beta = jax.nn.softmax(w, axis=0)
    atten = jnp.zeros((N, N), dtype=jnp.float32)
    for i in range(M):
        atten = atten.at[edges[i][0], edges[i][1]].add(alpha_list[i] * beta[i, 0])
    z_sem = (beta[None, :, :] * zp).sum(axis=1)
    out = z_sem @ Wp + bp
    return out, atten
